# Initial kernel scaffold; baseline (speedup 1.0000x reference)
import functools

import jax
import jax.numpy as jnp
from jax import lax
from jax.experimental import pallas as pl
from jax.experimental.pallas import tpu as pltpu
from jax.experimental.pallas import tpu_sc as plsc

B = 128
NODE = 714
NP = 768
E = 4096
VOCAB = 38733
EMB = 62

NSC = 2
NTILE = 16
SLAB = NP * NP // NTILE
EPT = E // NTILE

_f32 = jnp.float32
_i32 = jnp.int32

_mesh = plsc.VectorSubcoreMesh(core_axis_name="c", subcore_axis_name="s")


@functools.partial(
    pl.kernel,
    out_type=jax.ShapeDtypeStruct((B * NP * NP,), _f32),
    mesh=_mesh,
    scratch_types=[
        pltpu.VMEM((SLAB,), _f32),
        pltpu.VMEM((EPT,), _i32),
        pltpu.VMEM((EPT,), _i32),
        pltpu.VMEM((2, 128), _i32),
        pltpu.VMEM((128,), _f32),
        pltpu.VMEM((128,), _f32),
        pltpu.VMEM_SHARED((NP * NP,), _f32),
    ],
)
def _cnt_build(edges_hbm, out_hbm, stage, srcv, dstv, idxv, onesv, negv, acc):
    cid = lax.axis_index("c")
    sid = lax.axis_index("s")

    one16 = jnp.full((16,), 1.0, _f32)
    neg16 = jnp.full((16,), -1.0, _f32)
    zero16 = jnp.zeros((16,), _f32)
    for j in range(8):
        onesv[pl.ds(j * 16, 16)] = one16
        negv[pl.ds(j * 16, 16)] = neg16

    def _zero_body(i, carry):
        for j in range(4):
            stage[pl.ds(i * 64 + j * 16, 16)] = zero16
        return carry

    lax.fori_loop(0, SLAB // 64, _zero_body, 0)
    pltpu.sync_copy(stage, acc.at[pl.ds(sid * SLAB, SLAB)])
    plsc.subcore_barrier()

    def _round(r, carry):
        b = cid * (B // NSC) + r
        eoff = b * (2 * E)
        pltpu.sync_copy(edges_hbm.at[pl.ds(eoff + sid * EPT, EPT)], srcv)
        pltpu.sync_copy(edges_hbm.at[pl.ds(eoff + E + sid * EPT, EPT)], dstv)
        for i in range(EPT // 16):
            d = dstv[pl.ds(i * 16, 16)]
            s = srcv[pl.ds(i * 16, 16)]
            idxv[i // 8, pl.ds((i % 8) * 16, 16)] = d * NP + s
        for j in range(2):
            pltpu.sync_copy(onesv, acc.at[idxv.at[j]], add=True)
        plsc.subcore_barrier()
        pltpu.sync_copy(acc.at[pl.ds(sid * SLAB, SLAB)], stage)
        pltpu.sync_copy(stage, out_hbm.at[pl.ds(b * NP * NP + sid * SLAB, SLAB)])
        plsc.subcore_barrier()
        for j in range(2):
            pltpu.sync_copy(negv, acc.at[idxv.at[j]], add=True)
        plsc.subcore_barrier()
        return carry

    lax.fori_loop(0, B // NSC, _round, 0)


@functools.partial(
    pl.kernel,
    out_type=[
        jax.ShapeDtypeStruct((B * NP, 64), _f32),
        jax.ShapeDtypeStruct((B * NP, 64), _f32),
    ],
    mesh=_mesh,
    scratch_types=[
        pltpu.VMEM((NP // 128, 128), _i32),
        pltpu.VMEM((NP, 64), _f32),
        pltpu.SemaphoreType.DMA,
    ],
)
def _emb_gather(embp_hbm, id0_hbm, id1_hbm, xe0_hbm, xe1_hbm, idxc, rows, sem):
    cid = lax.axis_index("c")
    sid = lax.axis_index("s")
    wid = sid * NSC + cid
    bpw = B // (NSC * NTILE)
    nch = NP // 128

    def _one(idn_hbm, xen_hbm, b):
        pltpu.sync_copy(idn_hbm.at[pl.ds(b * nch, nch)], idxc)
        descs = [
            pltpu.async_copy(
                embp_hbm.at[idxc.at[c]], rows.at[pl.ds(c * 128, 128)], sem
            )
            for c in range(nch)
        ]
        for dsc in descs:
            dsc.wait()
        pltpu.sync_copy(rows, xen_hbm.at[pl.ds(b * NP, NP)])

    def _body(k, carry):
        b = wid * bpw + k
        _one(id0_hbm, xe0_hbm, b)
        _one(id1_hbm, xe1_hbm, b)
        return carry

    lax.fori_loop(0, bpw, _body, 0)


def _lrelu(t):
    return jnp.where(t >= 0.0, t, 0.01 * t)


def _tc_body(cnt_ref, xe0_ref, xe1_ref, flp_ref,
             w1a_ref, w1b_ref, w1c_ref, b1_ref, w2_ref, b2_ref,
             w3_ref, b3_ref, w4_ref, b4_ref,
             wf1t_ref, bf1_ref, wf2t_ref, bf2_ref, out_ref):
    cnt = cnt_ref[0]
    deg = 1.0 + jnp.sum(cnt, axis=1, keepdims=True)
    dinv = lax.rsqrt(deg)

    z = (jnp.dot(xe0_ref[0], w1a_ref[...], preferred_element_type=_f32)
         + jnp.dot(xe1_ref[0], w1b_ref[...], preferred_element_type=_f32)
         + jnp.dot(flp_ref[0], w1c_ref[...], preferred_element_type=_f32))
    rows = lax.broadcasted_iota(_i32, (NP, 1), 0)
    z = jnp.where(rows < NODE, z, 0.0)

    def agg(zz):
        zd = dinv * zz
        return dinv * (jnp.dot(cnt, zd, preferred_element_type=_f32) + zd)

    t = agg(z) + b1_ref[...]
    f = _lrelu(t)
    t = agg(jnp.dot(f, w2_ref[...], preferred_element_type=_f32)) + b2_ref[...]
    f = _lrelu(t) + t
    t = agg(jnp.dot(f, w3_ref[...], preferred_element_type=_f32)) + b3_ref[...]
    f = _lrelu(t) + t
    t = agg(jnp.dot(f, w4_ref[...], preferred_element_type=_f32)) + b4_ref[...]
    f4 = _lrelu(t)
    hcol = jnp.maximum(
        jnp.dot(wf1t_ref[...], f4, preferred_element_type=_f32) + bf1_ref[...], 0.0)
    ocol = jnp.maximum(
        jnp.dot(wf2t_ref[...], hcol, preferred_element_type=_f32) + bf2_ref[...], 0.0)
    out_ref[...] = ocol[None]


def _full(shape):
    return pl.BlockSpec(shape, lambda b: (0,) * len(shape))


_tc_net = pl.pallas_call(
    _tc_body,
    grid=(B,),
    in_specs=[
        pl.BlockSpec((1, NP, NP), lambda b: (b, 0, 0)),
        pl.BlockSpec((1, NP, 64), lambda b: (b, 0, 0)),
        pl.BlockSpec((1, NP, 64), lambda b: (b, 0, 0)),
        pl.BlockSpec((1, NP, 8), lambda b: (b, 0, 0)),
        _full((64, 32)), _full((64, 32)), _full((8, 32)), _full((1, 32)),
        _full((32, 32)), _full((1, 32)),
        _full((32, 32)), _full((1, 32)),
        _full((32, 8)), _full((1, 8)),
        _full((128, NP)), _full((128, 1)), _full((128, 128)), _full((128, 1)),
    ],
    out_specs=pl.BlockSpec((1, 128, 8), lambda b: (b, 0, 0)),
    out_shape=jax.ShapeDtypeStruct((B, 128, 8), _f32),
    compiler_params=pltpu.CompilerParams(
        dimension_semantics=("arbitrary",),
    ),
)


def kernel(feature, edges, emb_table, W1, b1, W2, b2, W3, b3, W4, b4,
           Wf1, bf1, Wf2, bf2):
    ids = feature[:, :, 0:2].astype(_i32)
    idsp = jnp.pad(ids, ((0, 0), (0, NP - NODE), (0, 0)))
    id0 = idsp[:, :, 0].reshape(B * NP // 128, 128)
    id1 = idsp[:, :, 1].reshape(B * NP // 128, 128)
    flp = jnp.pad(feature[:, :, 2:5], ((0, 0), (0, NP - NODE), (0, 5)))
    embp = jnp.pad(emb_table, ((0, 0), (0, 64 - EMB)))
    edges_flat = edges.reshape(B * 2 * E)

    w1a = jnp.pad(W1[0:EMB], ((0, 64 - EMB), (0, 0)))
    w1b = jnp.pad(W1[EMB:2 * EMB], ((0, 64 - EMB), (0, 0)))
    w1c = jnp.pad(W1[2 * EMB:], ((0, 5), (0, 0)))
    w4p = jnp.pad(W4, ((0, 0), (0, 7)))
    b4p = jnp.pad(b4, (0, 7)).reshape(1, 8)
    wf1t = jnp.pad(Wf1, ((0, NP - NODE), (0, 0))).T
    wf2t = Wf2.T

    cnt_flat = _cnt_build(edges_flat)
    xe0, xe1 = _emb_gather(embp, id0, id1)

    out_full = _tc_net(
        cnt_flat.reshape(B, NP, NP),
        xe0.reshape(B, NP, 64),
        xe1.reshape(B, NP, 64),
        flp,
        w1a, w1b, w1c, b1.reshape(1, 32),
        W2, b2.reshape(1, 32),
        W3, b3.reshape(1, 32),
        w4p, b4p,
        wf1t, bf1.reshape(128, 1), wf2t, bf2.reshape(128, 1),
    )
    return out_full[:, :, 0]


# speedup vs baseline: 14.2804x; 14.2804x over previous
import functools

import jax
import jax.numpy as jnp
from jax import lax
from jax.experimental import pallas as pl
from jax.experimental.pallas import tpu as pltpu
from jax.experimental.pallas import tpu_sc as plsc

B = 128
NODE = 714
NP = 768
E = 4096
VOCAB = 38733
EMB = 62

NSC = 2
NTILE = 16
SLAB = NP * NP // NTILE
EPT = E // NTILE

_f32 = jnp.float32
_i32 = jnp.int32


def _cnt_build_body(edges_hbm, out_hbm, stage, srcv, dstv, idxv, onesv, negv, acc):
    cid = lax.axis_index("c")
    sid = lax.axis_index("s")

    one16 = jnp.full((16,), 1.0, _f32)
    neg16 = jnp.full((16,), -1.0, _f32)
    zero16 = jnp.zeros((16,), _f32)
    for j in range(8):
        onesv[pl.ds(j * 16, 16)] = one16
        negv[pl.ds(j * 16, 16)] = neg16

    def _zero_body(i, carry):
        for j in range(4):
            stage[pl.ds(i * 64 + j * 16, 16)] = zero16
        return carry

    lax.fori_loop(0, SLAB // 64, _zero_body, 0)
    pltpu.sync_copy(stage, acc.at[pl.ds(sid * SLAB, SLAB)])
    plsc.subcore_barrier()

    def _round(r, carry):
        b = cid * (B // NSC) + r
        eoff = b * (2 * E)
        pltpu.sync_copy(edges_hbm.at[pl.ds(eoff + sid * EPT, EPT)], srcv)
        pltpu.sync_copy(edges_hbm.at[pl.ds(eoff + E + sid * EPT, EPT)], dstv)
        for i in range(EPT // 16):
            d = dstv[pl.ds(i * 16, 16)]
            s = srcv[pl.ds(i * 16, 16)]
            idxv[i // 8, pl.ds((i % 8) * 16, 16)] = d * NP + s
        for j in range(2):
            pltpu.sync_copy(onesv, acc.at[idxv.at[j]], add=True)
        plsc.subcore_barrier()
        pltpu.sync_copy(acc.at[pl.ds(sid * SLAB, SLAB)], stage)
        pltpu.sync_copy(stage, out_hbm.at[pl.ds(b * NP * NP + sid * SLAB, SLAB)])
        plsc.subcore_barrier()
        for j in range(2):
            pltpu.sync_copy(negv, acc.at[idxv.at[j]], add=True)
        plsc.subcore_barrier()
        return carry

    lax.fori_loop(0, B // NSC, _round, 0)


def _emb_gather_body(embp_hbm, id0_hbm, id1_hbm, xe0_hbm, xe1_hbm, idxc, rows, sem):
    cid = lax.axis_index("c")
    sid = lax.axis_index("s")
    wid = sid * NSC + cid
    bpw = B // (NSC * NTILE)
    nch = NP // 128

    def _one(idn_hbm, xen_hbm, b):
        pltpu.sync_copy(idn_hbm.at[pl.ds(b * NP, NP)], idxc)
        descs = [
            pltpu.async_copy(
                embp_hbm.at[idxc.at[pl.ds(c * 128, 128)]],
                rows.at[pl.ds(c * 128, 128)], sem
            )
            for c in range(nch)
        ]
        for dsc in descs:
            dsc.wait()
        pltpu.sync_copy(rows, xen_hbm.at[pl.ds(b * NP, NP)])

    def _body(k, carry):
        b = wid * bpw + k
        _one(id0_hbm, xe0_hbm, b)
        _one(id1_hbm, xe1_hbm, b)
        return carry

    lax.fori_loop(0, bpw, _body, 0)


@functools.cache
def _get_sc_kernels():
    mesh = plsc.VectorSubcoreMesh(core_axis_name="c", subcore_axis_name="s")
    cnt_build = pl.kernel(
        _cnt_build_body,
        out_type=jax.ShapeDtypeStruct((B * NP * NP,), _f32),
        mesh=mesh,
        scratch_types=[
            pltpu.VMEM((SLAB,), _f32),
            pltpu.VMEM((EPT,), _i32),
            pltpu.VMEM((EPT,), _i32),
            pltpu.VMEM((2, 128), _i32),
            pltpu.VMEM((128,), _f32),
            pltpu.VMEM((128,), _f32),
            pltpu.VMEM_SHARED((NP * NP,), _f32),
        ],
    )
    emb_gather = pl.kernel(
        _emb_gather_body,
        out_type=[
            jax.ShapeDtypeStruct((B * NP, 128), _f32),
            jax.ShapeDtypeStruct((B * NP, 128), _f32),
        ],
        mesh=mesh,
        scratch_types=[
            pltpu.VMEM((NP,), _i32),
            pltpu.VMEM((NP, 128), _f32),
            pltpu.SemaphoreType.DMA,
        ],
    )
    return cnt_build, emb_gather


def _lrelu(t):
    return jnp.where(t >= 0.0, t, 0.01 * t)


def _tc_body(cnt_ref, xe0_ref, xe1_ref, flp_ref,
             w1a_ref, w1b_ref, w1c_ref, b1_ref, w2_ref, b2_ref,
             w3_ref, b3_ref, w4_ref, b4_ref,
             wf1t_ref, bf1_ref, wf2t_ref, bf2_ref, out_ref):
    cnt = cnt_ref[0]
    deg = 1.0 + jnp.sum(cnt, axis=1, keepdims=True)
    dinv = lax.rsqrt(deg)

    z = (jnp.dot(xe0_ref[0], w1a_ref[...], preferred_element_type=_f32)
         + jnp.dot(xe1_ref[0], w1b_ref[...], preferred_element_type=_f32)
         + jnp.dot(flp_ref[0], w1c_ref[...], preferred_element_type=_f32))
    rows = lax.broadcasted_iota(_i32, (NP, 1), 0)
    z = jnp.where(rows < NODE, z, 0.0)

    def agg(zz):
        zd = dinv * zz
        return dinv * (jnp.dot(cnt, zd, preferred_element_type=_f32) + zd)

    t = agg(z) + b1_ref[...]
    f = _lrelu(t)
    t = agg(jnp.dot(f, w2_ref[...], preferred_element_type=_f32)) + b2_ref[...]
    f = _lrelu(t) + t
    t = agg(jnp.dot(f, w3_ref[...], preferred_element_type=_f32)) + b3_ref[...]
    f = _lrelu(t) + t
    t = agg(jnp.dot(f, w4_ref[...], preferred_element_type=_f32)) + b4_ref[...]
    f4 = _lrelu(t)
    hcol = jnp.maximum(
        jnp.dot(wf1t_ref[...], f4, preferred_element_type=_f32) + bf1_ref[...], 0.0)
    ocol = jnp.maximum(
        jnp.dot(wf2t_ref[...], hcol, preferred_element_type=_f32) + bf2_ref[...], 0.0)
    out_ref[...] = ocol[None]


def _full(shape):
    return pl.BlockSpec(shape, lambda b: (0,) * len(shape))


_tc_net = pl.pallas_call(
    _tc_body,
    grid=(B,),
    in_specs=[
        pl.BlockSpec((1, NP, NP), lambda b: (b, 0, 0)),
        pl.BlockSpec((1, NP, 128), lambda b: (b, 0, 0)),
        pl.BlockSpec((1, NP, 128), lambda b: (b, 0, 0)),
        pl.BlockSpec((1, NP, 8), lambda b: (b, 0, 0)),
        _full((128, 32)), _full((128, 32)), _full((8, 32)), _full((1, 32)),
        _full((32, 32)), _full((1, 32)),
        _full((32, 32)), _full((1, 32)),
        _full((32, 8)), _full((1, 8)),
        _full((128, NP)), _full((128, 1)), _full((128, 128)), _full((128, 1)),
    ],
    out_specs=pl.BlockSpec((1, 128, 8), lambda b: (b, 0, 0)),
    out_shape=jax.ShapeDtypeStruct((B, 128, 8), _f32),
    compiler_params=pltpu.CompilerParams(
        dimension_semantics=("arbitrary",),
    ),
)


def kernel(feature, edges, emb_table, W1, b1, W2, b2, W3, b3, W4, b4,
           Wf1, bf1, Wf2, bf2):
    ids = feature[:, :, 0:2].astype(_i32)
    idsp = jnp.pad(ids, ((0, 0), (0, NP - NODE), (0, 0)))
    id0 = idsp[:, :, 0].reshape(B * NP)
    id1 = idsp[:, :, 1].reshape(B * NP)
    flp = jnp.pad(feature[:, :, 2:5], ((0, 0), (0, NP - NODE), (0, 5)))
    embp = jnp.pad(emb_table, ((0, 0), (0, 128 - EMB)))
    edges_flat = edges.reshape(B * 2 * E)

    w1a = jnp.pad(W1[0:EMB], ((0, 128 - EMB), (0, 0)))
    w1b = jnp.pad(W1[EMB:2 * EMB], ((0, 128 - EMB), (0, 0)))
    w1c = jnp.pad(W1[2 * EMB:], ((0, 5), (0, 0)))
    w4p = jnp.pad(W4, ((0, 0), (0, 7)))
    b4p = jnp.pad(b4, (0, 7)).reshape(1, 8)
    wf1t = jnp.pad(Wf1, ((0, NP - NODE), (0, 0))).T
    wf2t = Wf2.T

    cnt_build, emb_gather = _get_sc_kernels()
    cnt_flat = cnt_build(edges_flat)
    xe0, xe1 = emb_gather(embp, id0, id1)

    out_full = _tc_net(
        cnt_flat.reshape(B, NP, NP),
        xe0.reshape(B, NP, 128),
        xe1.reshape(B, NP, 128),
        flp,
        w1a, w1b, w1c, b1.reshape(1, 32),
        W2, b2.reshape(1, 32),
        W3, b3.reshape(1, 32),
        w4p, b4p,
        wf1t, bf1.reshape(128, 1), wf2t, bf2.reshape(128, 1),
    )
    return out_full[:, :, 0]


# speedup vs baseline: 15.3402x; 1.0742x over previous
import functools

import jax
import jax.numpy as jnp
from jax import lax
from jax.experimental import pallas as pl
from jax.experimental.pallas import tpu as pltpu
from jax.experimental.pallas import tpu_sc as plsc

B = 128
NODE = 714
NP = 768
NPNP = NP * NP
E = 4096
VOCAB = 38733
EMB = 62

NSC = 2
NTILE = 16
BPSC = B // NSC
SLAB = NPNP // NTILE
EPT = E // NTILE
ZCH = 4608

_f32 = jnp.float32
_bf16 = jnp.bfloat16
_i32 = jnp.int32


def _cnt_build_body(edges_hbm, out_hbm,
                    esrc, edst, zbuf, idxv, onesv, negv, esem, wsem0, wsem1,
                    ssem, acc):
    cid = lax.axis_index("c")
    sid = lax.axis_index("s")

    one16 = jnp.full((16,), 1.0, _f32)
    neg16 = jnp.full((16,), -1.0, _f32)
    zero16 = jnp.zeros((16,), _f32)
    for j in range(8):
        onesv[pl.ds(j * 16, 16)] = one16
        negv[pl.ds(j * 16, 16)] = neg16

    base = cid * BPSC * 2 * E
    eds = []
    for r in range(BPSC):
        eoff = base + r * 2 * E + sid * EPT
        eds.append(pltpu.async_copy(
            edges_hbm.at[pl.ds(eoff, EPT)], esrc.at[r], esem))
        eds.append(pltpu.async_copy(
            edges_hbm.at[pl.ds(eoff + E, EPT)], edst.at[r], esem))

    def _zero_body(i, carry):
        for j in range(4):
            zbuf[pl.ds(i * 64 + j * 16, 16)] = zero16
        return carry

    lax.fori_loop(0, ZCH // 64, _zero_body, 0)
    for p in range(2):
        for k in range(SLAB // ZCH):
            pltpu.sync_copy(
                zbuf, acc.at[pl.ds(p * NPNP + sid * SLAB + k * ZCH, ZCH)])

    for k in range(4):
        for j in range(8):
            idxv[k, pl.ds(j * 16, 16)] = (
                2 * NPNP + j * 16 + lax.iota(_i32, 16))
    pltpu.async_copy(acc.at[pl.ds(sid * SLAB, SLAB)],
                     out_hbm.at[pl.ds(cid * BPSC * NPNP + sid * SLAB, SLAB)],
                     wsem0)
    pltpu.async_copy(acc.at[pl.ds(NPNP + sid * SLAB, SLAB)],
                     out_hbm.at[pl.ds((cid * BPSC + 1) * NPNP + sid * SLAB, SLAB)],
                     wsem1)
    for dsc in eds:
        dsc.wait()
    plsc.subcore_barrier()

    wsems = (wsem0, wsem1)

    def _pair(rr, carry):
        for slot in range(2):
            r = 2 * rr + slot
            b = cid * BPSC + r
            pltpu.make_async_copy(
                acc.at[pl.ds(slot * NPNP + sid * SLAB, SLAB)],
                out_hbm.at[pl.ds(b * NPNP + sid * SLAB, SLAB)],
                wsems[slot]).wait()
            plsc.subcore_barrier()
            for j in range(2):
                pltpu.sync_copy(negv, acc.at[idxv.at[2 * slot + j]], add=True)
            plsc.subcore_barrier()
            for i in range(EPT // 16):
                d = edst[r, pl.ds(i * 16, 16)]
                s = esrc[r, pl.ds(i * 16, 16)]
                idxv[2 * slot + i // 8, pl.ds((i % 8) * 16, 16)] = (
                    d * NP + s + slot * NPNP)
            for j in range(2):
                pltpu.sync_copy(onesv, acc.at[idxv.at[2 * slot + j]], add=True)
            plsc.subcore_barrier()
            pltpu.async_copy(acc.at[pl.ds(slot * NPNP + sid * SLAB, SLAB)],
                             out_hbm.at[pl.ds(b * NPNP + sid * SLAB, SLAB)],
                             wsems[slot])
        return carry

    lax.fori_loop(0, BPSC // 2, _pair, 0)

    for slot in range(2):
        b = cid * BPSC + BPSC - 2 + slot
        pltpu.make_async_copy(
            acc.at[pl.ds(slot * NPNP + sid * SLAB, SLAB)],
            out_hbm.at[pl.ds(b * NPNP + sid * SLAB, SLAB)],
            wsems[slot]).wait()


def _emb_gather_body(embp_hbm, id0_hbm, id1_hbm, xe0_hbm, xe1_hbm, idxc, rows, sem):
    cid = lax.axis_index("c")
    sid = lax.axis_index("s")
    wid = sid * NSC + cid
    bpw = B // (NSC * NTILE)
    nch = NP // 128

    def _one(idn_hbm, xen_hbm, b):
        pltpu.sync_copy(idn_hbm.at[pl.ds(b * NP, NP)], idxc)
        descs = [
            pltpu.async_copy(
                embp_hbm.at[idxc.at[pl.ds(c * 128, 128)]],
                rows.at[pl.ds(c * 128, 128)], sem
            )
            for c in range(nch)
        ]
        for dsc in descs:
            dsc.wait()
        pltpu.sync_copy(rows, xen_hbm.at[pl.ds(b * NP, NP)])

    def _body(k, carry):
        b = wid * bpw + k
        _one(id0_hbm, xe0_hbm, b)
        _one(id1_hbm, xe1_hbm, b)
        return carry

    lax.fori_loop(0, bpw, _body, 0)


@functools.cache
def _get_sc_kernels():
    mesh = plsc.VectorSubcoreMesh(core_axis_name="c", subcore_axis_name="s")
    cnt_build = pl.kernel(
        _cnt_build_body,
        out_type=jax.ShapeDtypeStruct((B * NPNP,), _f32),
        mesh=mesh,
        scratch_types=[
            pltpu.VMEM((BPSC, EPT), _i32),
            pltpu.VMEM((BPSC, EPT), _i32),
            pltpu.VMEM((ZCH,), _f32),
            pltpu.VMEM((4, 128), _i32),
            pltpu.VMEM((128,), _f32),
            pltpu.VMEM((128,), _f32),
            pltpu.SemaphoreType.DMA,
            pltpu.SemaphoreType.DMA,
            pltpu.SemaphoreType.DMA,
            pltpu.SemaphoreType.DMA,
            pltpu.VMEM_SHARED((2 * NPNP + 128,), _f32),
        ],
    )
    emb_gather = pl.kernel(
        _emb_gather_body,
        out_type=[
            jax.ShapeDtypeStruct((B * NP, 128), _f32),
            jax.ShapeDtypeStruct((B * NP, 128), _f32),
        ],
        mesh=mesh,
        scratch_types=[
            pltpu.VMEM((NP,), _i32),
            pltpu.VMEM((NP, 128), _f32),
            pltpu.SemaphoreType.DMA,
        ],
    )
    return cnt_build, emb_gather


def _lrelu(t):
    return jnp.where(t >= 0.0, t, 0.01 * t)


def _tc_body(cnt_ref, xe0_ref, xe1_ref, flp_ref,
             w1a_ref, w1b_ref, w1c_ref, b1_ref, w2_ref, b2_ref,
             w3_ref, b3_ref, w4_ref, b4_ref,
             wf1t_ref, bf1_ref, wf2t_ref, bf2_ref, out_ref):
    cnt = cnt_ref[0]
    deg = 1.0 + jnp.sum(cnt, axis=1, keepdims=True)
    dinv = lax.rsqrt(deg)
    cbf = cnt.astype(_bf16)

    z = (jnp.dot(xe0_ref[0], w1a_ref[...], preferred_element_type=_f32)
         + jnp.dot(xe1_ref[0], w1b_ref[...], preferred_element_type=_f32)
         + jnp.dot(flp_ref[0], w1c_ref[...], preferred_element_type=_f32))
    rows = lax.broadcasted_iota(_i32, (NP, 1), 0)
    z = jnp.where(rows < NODE, z, 0.0)

    def agg(zz):
        zd = dinv * zz
        mm = jnp.dot(cnt, zd, preferred_element_type=_f32)
        return dinv * (mm + zd)

    t = agg(z) + b1_ref[...]
    f = _lrelu(t)
    t = agg(jnp.dot(f, w2_ref[...], preferred_element_type=_f32)) + b2_ref[...]
    f = _lrelu(t) + t
    t = agg(jnp.dot(f, w3_ref[...], preferred_element_type=_f32)) + b3_ref[...]
    f = _lrelu(t) + t
    t = agg(jnp.dot(f, w4_ref[...], preferred_element_type=_f32)) + b4_ref[...]
    f4 = _lrelu(t)
    hcol = jnp.maximum(
        jnp.dot(wf1t_ref[...], f4, preferred_element_type=_f32) + bf1_ref[...], 0.0)
    ocol = jnp.maximum(
        jnp.dot(wf2t_ref[...], hcol, preferred_element_type=_f32) + bf2_ref[...], 0.0)
    out_ref[...] = ocol[None]


def _full(shape):
    return pl.BlockSpec(shape, lambda b: (0,) * len(shape))


_tc_net = pl.pallas_call(
    _tc_body,
    grid=(B,),
    in_specs=[
        pl.BlockSpec((1, NP, NP), lambda b: (b, 0, 0)),
        pl.BlockSpec((1, NP, 128), lambda b: (b, 0, 0)),
        pl.BlockSpec((1, NP, 128), lambda b: (b, 0, 0)),
        pl.BlockSpec((1, NP, 8), lambda b: (b, 0, 0)),
        _full((128, 32)), _full((128, 32)), _full((8, 32)), _full((1, 32)),
        _full((32, 32)), _full((1, 32)),
        _full((32, 32)), _full((1, 32)),
        _full((32, 8)), _full((1, 8)),
        _full((128, NP)), _full((128, 1)), _full((128, 128)), _full((128, 1)),
    ],
    out_specs=pl.BlockSpec((1, 128, 8), lambda b: (b, 0, 0)),
    out_shape=jax.ShapeDtypeStruct((B, 128, 8), _f32),
    compiler_params=pltpu.CompilerParams(
        dimension_semantics=("arbitrary",),
    ),
)


def kernel(feature, edges, emb_table, W1, b1, W2, b2, W3, b3, W4, b4,
           Wf1, bf1, Wf2, bf2):
    ids = feature[:, :, 0:2].astype(_i32)
    idsp = jnp.pad(ids, ((0, 0), (0, NP - NODE), (0, 0)))
    id0 = idsp[:, :, 0].reshape(B * NP)
    id1 = idsp[:, :, 1].reshape(B * NP)
    flp = jnp.pad(feature[:, :, 2:5], ((0, 0), (0, NP - NODE), (0, 5)))
    embp = jnp.pad(emb_table, ((0, 0), (0, 128 - EMB)))
    edges_flat = edges.reshape(B * 2 * E)

    w1a = jnp.pad(W1[0:EMB], ((0, 128 - EMB), (0, 0)))
    w1b = jnp.pad(W1[EMB:2 * EMB], ((0, 128 - EMB), (0, 0)))
    w1c = jnp.pad(W1[2 * EMB:], ((0, 5), (0, 0)))
    w4p = jnp.pad(W4, ((0, 0), (0, 7)))
    b4p = jnp.pad(b4, (0, 7)).reshape(1, 8)
    wf1t = jnp.pad(Wf1, ((0, NP - NODE), (0, 0))).T
    wf2t = Wf2.T

    cnt_build, emb_gather = _get_sc_kernels()
    cnt_flat = cnt_build(edges_flat)
    xe0, xe1 = emb_gather(embp, id0, id1)

    out_full = _tc_net(
        cnt_flat.reshape(B, NP, NP),
        xe0.reshape(B, NP, 128),
        xe1.reshape(B, NP, 128),
        flp,
        w1a, w1b, w1c, b1.reshape(1, 32),
        W2, b2.reshape(1, 32),
        W3, b3.reshape(1, 32),
        w4p, b4p,
        wf1t, bf1.reshape(128, 1), wf2t, bf2.reshape(128, 1),
    )
    return out_full[:, :, 0]
